# Initial kernel scaffold; baseline (speedup 1.0000x reference)
import jax
import jax.numpy as jnp
from jax.experimental import pallas as pl


def kernel(x, edge_index, W1, b1, W2, b2):
    raise NotImplementedError("write your pallas kernel here")


# speedup vs baseline: 23.8985x; 23.8985x over previous
import functools

import jax
import jax.numpy as jnp
from jax import lax
from jax.experimental import pallas as pl
from jax.experimental.pallas import tpu as pltpu
from jax.experimental.pallas import tpu_sc as plsc

N = 10000
NP = 10240
E = 320000
IN_DIM = 128
HID = 64
OUT = 32

NC = 2
NS = 16
NW = NC * NS
EB = 80
EROWS = E // EB
ROWS_PER_TILE = EROWS // NW
IDX_CH = 25
NODES_PER_TILE = NP // NS

ROW_BLK = 512


def _mesh():
  return plsc.VectorSubcoreMesh(core_axis_name="c", subcore_axis_name="s")


def _deg_body(dst_hbm, degp_hbm, dst_v, deg_v):
  c = lax.axis_index("c")
  s = lax.axis_index("s")
  wid = s * NC + c
  per_tile = E // NW
  ch = 2000

  zeros16 = jnp.zeros((16,), jnp.float32)

  def zero_body(i, carry):
    deg_v[pl.ds(i * 16, 16)] = zeros16
    return carry

  lax.fori_loop(0, NP // 16, zero_body, 0)

  ones16 = jnp.ones((16,), jnp.float32)

  def chunk_body(ci, carry):
    base = wid * per_tile + ci * ch
    pltpu.sync_copy(dst_hbm.at[pl.ds(base, ch)], dst_v)

    def inner(i, c2):
      idx = dst_v[pl.ds(i * 16, 16)]
      plsc.addupdate_scatter(deg_v, [idx], ones16)
      return c2

    lax.fori_loop(0, ch // 16, inner, 0)
    return carry

  lax.fori_loop(0, per_tile // ch, chunk_body, 0)
  pltpu.sync_copy(deg_v, degp_hbm.at[wid])


_SC_PARAMS = pltpu.CompilerParams(needs_layout_passes=False,
                                  use_tc_tiling_on_sc=False)

_deg_call = pl.kernel(
    _deg_body,
    out_type=jax.ShapeDtypeStruct((NW, NP), jnp.float32),
    mesh=_mesh(),
    compiler_params=_SC_PARAMS,
    scratch_types=[
        pltpu.VMEM((2000,), jnp.int32),
        pltpu.VMEM((NP,), jnp.float32),
    ],
)


def _agg_body(hs_hbm, src_hbm, dst_hbm, aggp_hbm, idxs_v, idxd_v, rows_v,
              zero_v, agg_sp, *, dim):
  c = lax.axis_index("c")
  s = lax.axis_index("s")
  wid = s * NC + c

  zeros16 = jnp.zeros((16,), jnp.float32)
  lanes_per_row = dim // 16

  def zero_body(i, carry):
    r = i // lanes_per_row
    k = (i % lanes_per_row) * 16
    zero_v[r, pl.ds(k, 16)] = zeros16
    return carry

  lax.fori_loop(0, NODES_PER_TILE * lanes_per_row, zero_body, 0)
  pltpu.sync_copy(zero_v, agg_sp.at[pl.ds(s * NODES_PER_TILE, NODES_PER_TILE)])
  plsc.subcore_barrier()

  def outer(oi, carry):
    pltpu.sync_copy(src_hbm.at[wid, oi], idxs_v)
    pltpu.sync_copy(dst_hbm.at[wid, oi], idxd_v)

    def inner(j, c2):
      pltpu.sync_copy(hs_hbm.at[idxs_v.at[j]], rows_v)
      pltpu.sync_copy(rows_v, agg_sp.at[idxd_v.at[j]], add=True)
      return c2

    lax.fori_loop(0, IDX_CH, inner, 0)
    return carry

  lax.fori_loop(0, ROWS_PER_TILE // IDX_CH, outer, 0)
  plsc.subcore_barrier()
  pltpu.sync_copy(
      agg_sp.at[pl.ds(s * NODES_PER_TILE, NODES_PER_TILE)],
      aggp_hbm.at[c, pl.ds(s * NODES_PER_TILE, NODES_PER_TILE)],
  )


def _make_agg(dim):
  return pl.kernel(
      functools.partial(_agg_body, dim=dim),
      out_type=jax.ShapeDtypeStruct((NC, NP, dim), jnp.float32),
      mesh=_mesh(),
      compiler_params=_SC_PARAMS,
      scratch_types=[
          pltpu.VMEM((IDX_CH, EB), jnp.int32),
          pltpu.VMEM((IDX_CH, EB), jnp.int32),
          pltpu.VMEM((EB, dim), jnp.float32),
          pltpu.VMEM((NODES_PER_TILE, dim), jnp.float32),
          pltpu.VMEM_SHARED((NP, dim), jnp.float32),
      ],
  )


_agg_hid = _make_agg(HID)
_agg_out = _make_agg(OUT)


def _tc_a_body(x_ref, w1_ref, degp_ref, hs1_ref, dinv_ref):
  degp = degp_ref[...]
  deg = jnp.sum(degp.T, axis=1, keepdims=True) + 1.0
  dinv = lax.rsqrt(deg)
  h = jnp.dot(x_ref[...], w1_ref[...], preferred_element_type=jnp.float32)
  hs1_ref[...] = h * dinv
  dinv_ref[...] = jnp.broadcast_to(dinv, dinv_ref.shape)


_tc_a = pl.pallas_call(
    _tc_a_body,
    grid=(NP // ROW_BLK,),
    in_specs=[
        pl.BlockSpec((ROW_BLK, IN_DIM), lambda j: (j, 0)),
        pl.BlockSpec((IN_DIM, HID), lambda j: (0, 0)),
        pl.BlockSpec((NW, ROW_BLK), lambda j: (0, j)),
    ],
    out_specs=[
        pl.BlockSpec((ROW_BLK, HID), lambda j: (j, 0)),
        pl.BlockSpec((ROW_BLK, 128), lambda j: (j, 0)),
    ],
    out_shape=[
        jax.ShapeDtypeStruct((NP, HID), jnp.float32),
        jax.ShapeDtypeStruct((NP, 128), jnp.float32),
    ],
)


def _tc_b_body(p_ref, hs1_ref, dinv_ref, b1_ref, w2_ref, hs2_ref):
  p = p_ref[...]
  dinv = dinv_ref[...]
  agg = p[0] + p[1] + hs1_ref[...]
  out1 = agg * dinv[:, :HID] + b1_ref[...]
  r = jnp.maximum(out1, 0.0)
  h2 = jnp.dot(r, w2_ref[...], preferred_element_type=jnp.float32)
  hs2_ref[...] = h2 * dinv[:, :OUT]


_tc_b = pl.pallas_call(
    _tc_b_body,
    grid=(NP // ROW_BLK,),
    in_specs=[
        pl.BlockSpec((NC, ROW_BLK, HID), lambda j: (0, j, 0)),
        pl.BlockSpec((ROW_BLK, HID), lambda j: (j, 0)),
        pl.BlockSpec((ROW_BLK, 128), lambda j: (j, 0)),
        pl.BlockSpec((1, HID), lambda j: (0, 0)),
        pl.BlockSpec((HID, OUT), lambda j: (0, 0)),
    ],
    out_specs=pl.BlockSpec((ROW_BLK, OUT), lambda j: (j, 0)),
    out_shape=jax.ShapeDtypeStruct((NP, OUT), jnp.float32),
)


def _tc_c_body(q_ref, hs2_ref, dinv_ref, b2_ref, out_ref):
  q = q_ref[...]
  dinv = dinv_ref[...]
  out_ref[...] = (q[0] + q[1] + hs2_ref[...]) * dinv[:, :OUT] + b2_ref[...]


_tc_c = pl.pallas_call(
    _tc_c_body,
    grid=(NP // ROW_BLK,),
    in_specs=[
        pl.BlockSpec((NC, ROW_BLK, OUT), lambda j: (0, j, 0)),
        pl.BlockSpec((ROW_BLK, OUT), lambda j: (j, 0)),
        pl.BlockSpec((ROW_BLK, 128), lambda j: (j, 0)),
        pl.BlockSpec((1, OUT), lambda j: (0, 0)),
    ],
    out_specs=pl.BlockSpec((ROW_BLK, OUT), lambda j: (j, 0)),
    out_shape=jax.ShapeDtypeStruct((NP, OUT), jnp.float32),
)


@jax.jit
def kernel(x, edge_index, W1, b1, W2, b2):
  ei = edge_index.astype(jnp.int32)
  n_ch = ROWS_PER_TILE // IDX_CH
  src2 = ei[0].reshape(NW, n_ch, IDX_CH, EB)
  dst1 = ei[1]
  dst2 = dst1.reshape(NW, n_ch, IDX_CH, EB)

  x_p = jnp.pad(x, ((0, NP - N), (0, 0)))
  degp = _deg_call(dst1)
  hs1, dinv = _tc_a(x_p, W1, degp)
  p = _agg_hid(hs1, src2, dst2)
  hs2 = _tc_b(p, hs1, dinv, b1.reshape(1, HID), W2)
  q = _agg_out(hs2, src2, dst2)
  return _tc_c(q, hs2, dinv, b2.reshape(1, OUT))[:N]


# speedup vs baseline: 43.1001x; 1.8035x over previous
import functools

import jax
import jax.numpy as jnp
from jax import lax
from jax.experimental import pallas as pl
from jax.experimental.pallas import tpu as pltpu
from jax.experimental.pallas import tpu_sc as plsc

N = 10000
NP = 10240
E = 320000
IN_DIM = 128
HID = 64
OUT = 32

NC = 2
NS = 16
NW = NC * NS
EB = 100
EROWS = E // EB
RPT = EROWS // NW
GRP = 5
NIT = RPT // (2 * GRP)
NBUF = 2 * GRP
NODES_PER_TILE = NP // NS

ROW_BLK = 512


def _mesh():
  return plsc.VectorSubcoreMesh(core_axis_name="c", subcore_axis_name="s")


def _deg_body(dst_hbm, degp_hbm, dst_v, deg_v):
  c = lax.axis_index("c")
  s = lax.axis_index("s")
  wid = s * NC + c
  per_tile = E // NW
  ch = 2000

  zeros16 = jnp.zeros((16,), jnp.float32)

  def zero_body(i, carry):
    deg_v[pl.ds(i * 16, 16)] = zeros16
    return carry

  lax.fori_loop(0, NP // 16, zero_body, 0)

  ones16 = jnp.ones((16,), jnp.float32)

  def chunk_body(ci, carry):
    base = wid * per_tile + ci * ch
    pltpu.sync_copy(dst_hbm.at[pl.ds(base, ch)], dst_v)

    def inner(i, c2):
      idx = dst_v[pl.ds(i * 16, 16)]
      plsc.addupdate_scatter(deg_v, [idx], ones16)
      return c2

    lax.fori_loop(0, ch // 16, inner, 0)
    return carry

  lax.fori_loop(0, per_tile // ch, chunk_body, 0)
  pltpu.sync_copy(deg_v, degp_hbm.at[wid])


_SC_PARAMS = pltpu.CompilerParams(needs_layout_passes=False,
                                  use_tc_tiling_on_sc=False)

_deg_call = pl.kernel(
    _deg_body,
    out_type=jax.ShapeDtypeStruct((NW, NP), jnp.float32),
    mesh=_mesh(),
    compiler_params=_SC_PARAMS,
    scratch_types=[
        pltpu.VMEM((2000,), jnp.int32),
        pltpu.VMEM((NP,), jnp.float32),
    ],
)


def _agg_body(hs_hbm, src_hbm, dst_hbm, aggp_hbm, idxs_v, idxd_v, rows_v,
              gsem_a, gsem_b, agg_sp, *, dim):
  c = lax.axis_index("c")
  s = lax.axis_index("s")
  wid = s * NC + c

  zeros16 = jnp.zeros((16,), jnp.float32)
  lanes_per_row = dim // 16

  def zero_body(i, carry):
    for l in range(lanes_per_row):
      rows_v[i, pl.ds(l * 16, 16)] = zeros16
    return carry

  lax.fori_loop(0, NODES_PER_TILE, zero_body, 0)
  pltpu.sync_copy(rows_v.at[pl.ds(0, NODES_PER_TILE)],
                  agg_sp.at[pl.ds(s * NODES_PER_TILE, NODES_PER_TILE)])
  pltpu.sync_copy(src_hbm.at[wid], idxs_v)
  pltpu.sync_copy(dst_hbm.at[wid], idxd_v)
  plsc.subcore_barrier()

  def gather(j, b, sem):
    pltpu.async_copy(hs_hbm.at[idxs_v.at[j]], rows_v.at[pl.ds(b * EB, EB)], sem)

  def gwait(b, sem):
    pltpu.make_async_copy(hs_hbm.at[idxs_v.at[0]],
                          rows_v.at[pl.ds(b * EB, EB)], sem).wait()

  def scat(j, b):
    pltpu.sync_copy(rows_v.at[pl.ds(b * EB, EB)], agg_sp.at[idxd_v.at[j]],
                    add=True)

  for b in range(GRP):
    gather(b, b, gsem_a)

  def body(t, carry):
    g0 = 2 * t * GRP
    for b in range(GRP):
      gather(g0 + GRP + b, GRP + b, gsem_b)
    for b in range(GRP):
      gwait(b, gsem_a)
    for b in range(GRP):
      scat(g0 + b, b)
    @pl.when(t < NIT - 1)
    def _():
      for b in range(GRP):
        gather(g0 + 2 * GRP + b, b, gsem_a)
    for b in range(GRP):
      gwait(GRP + b, gsem_b)
    for b in range(GRP):
      scat(g0 + GRP + b, GRP + b)
    return carry

  lax.fori_loop(0, NIT, body, 0)
  plsc.subcore_barrier()
  pltpu.sync_copy(
      agg_sp.at[pl.ds(s * NODES_PER_TILE, NODES_PER_TILE)],
      aggp_hbm.at[c, pl.ds(s * NODES_PER_TILE, NODES_PER_TILE)],
  )


def _make_agg(dim):
  return pl.kernel(
      functools.partial(_agg_body, dim=dim),
      out_type=jax.ShapeDtypeStruct((NC, NP, dim), jnp.float32),
      mesh=_mesh(),
      compiler_params=_SC_PARAMS,
      scratch_types=[
          pltpu.VMEM((RPT, EB), jnp.int32),
          pltpu.VMEM((RPT, EB), jnp.int32),
          pltpu.VMEM((NBUF * EB, dim), jnp.float32),
          pltpu.SemaphoreType.DMA,
          pltpu.SemaphoreType.DMA,
          pltpu.VMEM_SHARED((NP, dim), jnp.float32),
      ],
  )


_agg_hid = _make_agg(HID)
_agg_out = _make_agg(OUT)


def _tc_a_body(x_ref, w1_ref, degp_ref, hs1_ref, dinv_ref):
  degp = degp_ref[...]
  deg = jnp.sum(degp.T, axis=1, keepdims=True) + 1.0
  dinv = lax.rsqrt(deg)
  h = jnp.dot(x_ref[...], w1_ref[...], preferred_element_type=jnp.float32)
  hs1_ref[...] = h * dinv
  dinv_ref[...] = jnp.broadcast_to(dinv, dinv_ref.shape)


_tc_a = pl.pallas_call(
    _tc_a_body,
    grid=(NP // ROW_BLK,),
    in_specs=[
        pl.BlockSpec((ROW_BLK, IN_DIM), lambda j: (j, 0)),
        pl.BlockSpec((IN_DIM, HID), lambda j: (0, 0)),
        pl.BlockSpec((NW, ROW_BLK), lambda j: (0, j)),
    ],
    out_specs=[
        pl.BlockSpec((ROW_BLK, HID), lambda j: (j, 0)),
        pl.BlockSpec((ROW_BLK, 128), lambda j: (j, 0)),
    ],
    out_shape=[
        jax.ShapeDtypeStruct((NP, HID), jnp.float32),
        jax.ShapeDtypeStruct((NP, 128), jnp.float32),
    ],
)


def _tc_b_body(p_ref, hs1_ref, dinv_ref, b1_ref, w2_ref, hs2_ref):
  p = p_ref[...]
  dinv = dinv_ref[...]
  agg = p[0] + p[1] + hs1_ref[...]
  out1 = agg * dinv[:, :HID] + b1_ref[...]
  r = jnp.maximum(out1, 0.0)
  h2 = jnp.dot(r, w2_ref[...], preferred_element_type=jnp.float32)
  hs2_ref[...] = h2 * dinv[:, :OUT]


_tc_b = pl.pallas_call(
    _tc_b_body,
    grid=(NP // ROW_BLK,),
    in_specs=[
        pl.BlockSpec((NC, ROW_BLK, HID), lambda j: (0, j, 0)),
        pl.BlockSpec((ROW_BLK, HID), lambda j: (j, 0)),
        pl.BlockSpec((ROW_BLK, 128), lambda j: (j, 0)),
        pl.BlockSpec((1, HID), lambda j: (0, 0)),
        pl.BlockSpec((HID, OUT), lambda j: (0, 0)),
    ],
    out_specs=pl.BlockSpec((ROW_BLK, OUT), lambda j: (j, 0)),
    out_shape=jax.ShapeDtypeStruct((NP, OUT), jnp.float32),
)


def _tc_c_body(q_ref, hs2_ref, dinv_ref, b2_ref, out_ref):
  q = q_ref[...]
  dinv = dinv_ref[...]
  out_ref[...] = (q[0] + q[1] + hs2_ref[...]) * dinv[:, :OUT] + b2_ref[...]


_tc_c = pl.pallas_call(
    _tc_c_body,
    grid=(NP // ROW_BLK,),
    in_specs=[
        pl.BlockSpec((NC, ROW_BLK, OUT), lambda j: (0, j, 0)),
        pl.BlockSpec((ROW_BLK, OUT), lambda j: (j, 0)),
        pl.BlockSpec((ROW_BLK, 128), lambda j: (j, 0)),
        pl.BlockSpec((1, OUT), lambda j: (0, 0)),
    ],
    out_specs=pl.BlockSpec((ROW_BLK, OUT), lambda j: (j, 0)),
    out_shape=jax.ShapeDtypeStruct((NP, OUT), jnp.float32),
)


@jax.jit
def kernel(x, edge_index, W1, b1, W2, b2):
  ei = edge_index.astype(jnp.int32)
  src2 = ei[0].reshape(NW, RPT, EB)
  dst1 = ei[1]
  dst2 = dst1.reshape(NW, RPT, EB)

  x_p = jnp.pad(x, ((0, NP - N), (0, 0)))
  degp = _deg_call(dst1)
  hs1, dinv = _tc_a(x_p, W1, degp)
  p = _agg_hid(hs1, src2, dst2)
  hs2 = _tc_b(p, hs1, dinv, b1.reshape(1, HID), W2)
  q = _agg_out(hs2, src2, dst2)
  return _tc_c(q, hs2, dinv, b2.reshape(1, OUT))[:N]


# speedup vs baseline: 44.7677x; 1.0387x over previous
import functools

import jax
import jax.numpy as jnp
from jax import lax
from jax.experimental import pallas as pl
from jax.experimental.pallas import tpu as pltpu
from jax.experimental.pallas import tpu_sc as plsc

N = 10000
NP = 10240
E = 320000
IN_DIM = 128
HID = 64
OUT = 32

NC = 2
NS = 16
NW = NC * NS
EB = 125
EROWS = E // EB
RPT = EROWS // NW
GRP = 4
NIT = RPT // (2 * GRP)
NBUF = 2 * GRP
NODES_PER_TILE = NP // NS

ROW_BLK = 512


def _mesh():
  return plsc.VectorSubcoreMesh(core_axis_name="c", subcore_axis_name="s")


def _deg_body(dst_hbm, degp_hbm, dst_v, deg_v):
  c = lax.axis_index("c")
  s = lax.axis_index("s")
  wid = s * NC + c
  per_tile = E // NW
  ch = 2000

  zeros16 = jnp.zeros((16,), jnp.float32)

  def zero_body(i, carry):
    deg_v[pl.ds(i * 16, 16)] = zeros16
    return carry

  lax.fori_loop(0, NP // 16, zero_body, 0)

  ones16 = jnp.ones((16,), jnp.float32)

  def chunk_body(ci, carry):
    base = wid * per_tile + ci * ch
    pltpu.sync_copy(dst_hbm.at[pl.ds(base, ch)], dst_v)

    def inner(i, c2):
      idx = dst_v[pl.ds(i * 16, 16)]
      plsc.addupdate_scatter(deg_v, [idx], ones16)
      return c2

    lax.fori_loop(0, ch // 16, inner, 0)
    return carry

  lax.fori_loop(0, per_tile // ch, chunk_body, 0)
  pltpu.sync_copy(deg_v, degp_hbm.at[wid])


_SC_PARAMS = pltpu.CompilerParams(needs_layout_passes=False,
                                  use_tc_tiling_on_sc=False)

_deg_call = pl.kernel(
    _deg_body,
    out_type=jax.ShapeDtypeStruct((NW, NP), jnp.float32),
    mesh=_mesh(),
    compiler_params=_SC_PARAMS,
    scratch_types=[
        pltpu.VMEM((2000,), jnp.int32),
        pltpu.VMEM((NP,), jnp.float32),
    ],
)


def _agg_body(hs_hbm, src_hbm, dst_hbm, aggp_hbm, idxs_v, idxd_v, rows_v,
              gsem_a, gsem_b, agg_sp, *, dim):
  c = lax.axis_index("c")
  s = lax.axis_index("s")
  wid = s * NC + c

  zeros16 = jnp.zeros((16,), jnp.float32)
  lanes_per_row = dim // 16

  def zero_body(i, carry):
    for l in range(lanes_per_row):
      rows_v[i, pl.ds(l * 16, 16)] = zeros16
    return carry

  pltpu.async_copy(src_hbm.at[wid], idxs_v, gsem_a)
  pltpu.async_copy(dst_hbm.at[wid], idxd_v, gsem_b)
  lax.fori_loop(0, NODES_PER_TILE, zero_body, 0)
  pltpu.sync_copy(rows_v.at[pl.ds(0, NODES_PER_TILE)],
                  agg_sp.at[pl.ds(s * NODES_PER_TILE, NODES_PER_TILE)])
  pltpu.make_async_copy(src_hbm.at[wid], idxs_v, gsem_a).wait()
  pltpu.make_async_copy(dst_hbm.at[wid], idxd_v, gsem_b).wait()
  plsc.subcore_barrier()

  def gather(j, b, sem):
    pltpu.async_copy(hs_hbm.at[idxs_v.at[j]], rows_v.at[pl.ds(b * EB, EB)], sem)

  def gwait(b, sem):
    pltpu.make_async_copy(hs_hbm.at[idxs_v.at[0]],
                          rows_v.at[pl.ds(b * EB, EB)], sem).wait()

  def scat(j, b):
    pltpu.sync_copy(rows_v.at[pl.ds(b * EB, EB)], agg_sp.at[idxd_v.at[j]],
                    add=True)

  for b in range(GRP):
    gather(b, b, gsem_a)

  def body(t, carry):
    g0 = 2 * t * GRP
    for b in range(GRP):
      gather(g0 + GRP + b, GRP + b, gsem_b)
    for b in range(GRP):
      gwait(b, gsem_a)
    for b in range(GRP):
      scat(g0 + b, b)
    @pl.when(t < NIT - 1)
    def _():
      for b in range(GRP):
        gather(g0 + 2 * GRP + b, b, gsem_a)
    for b in range(GRP):
      gwait(GRP + b, gsem_b)
    for b in range(GRP):
      scat(g0 + GRP + b, GRP + b)
    return carry

  lax.fori_loop(0, NIT, body, 0)
  plsc.subcore_barrier()
  pltpu.sync_copy(
      agg_sp.at[pl.ds(s * NODES_PER_TILE, NODES_PER_TILE)],
      aggp_hbm.at[c, pl.ds(s * NODES_PER_TILE, NODES_PER_TILE)],
  )


def _make_agg(dim):
  return pl.kernel(
      functools.partial(_agg_body, dim=dim),
      out_type=jax.ShapeDtypeStruct((NC, NP, dim), jnp.float32),
      mesh=_mesh(),
      compiler_params=_SC_PARAMS,
      scratch_types=[
          pltpu.VMEM((RPT, EB), jnp.int32),
          pltpu.VMEM((RPT, EB), jnp.int32),
          pltpu.VMEM((NBUF * EB, dim), jnp.float32),
          pltpu.SemaphoreType.DMA,
          pltpu.SemaphoreType.DMA,
          pltpu.VMEM_SHARED((NP, dim), jnp.float32),
      ],
  )


_agg_hid = _make_agg(HID)
_agg_out = _make_agg(OUT)


def _tc_a_body(x_ref, w1_ref, degp_ref, hs1_ref, dinv_ref):
  degp = degp_ref[...]
  deg = jnp.sum(degp.T, axis=1, keepdims=True) + 1.0
  dinv = lax.rsqrt(deg)
  h = jnp.dot(x_ref[...], w1_ref[...], preferred_element_type=jnp.float32)
  hs1_ref[...] = h * dinv
  dinv_ref[...] = jnp.broadcast_to(dinv, dinv_ref.shape)


_tc_a = pl.pallas_call(
    _tc_a_body,
    grid=(NP // ROW_BLK,),
    in_specs=[
        pl.BlockSpec((ROW_BLK, IN_DIM), lambda j: (j, 0)),
        pl.BlockSpec((IN_DIM, HID), lambda j: (0, 0)),
        pl.BlockSpec((NW, ROW_BLK), lambda j: (0, j)),
    ],
    out_specs=[
        pl.BlockSpec((ROW_BLK, HID), lambda j: (j, 0)),
        pl.BlockSpec((ROW_BLK, 128), lambda j: (j, 0)),
    ],
    out_shape=[
        jax.ShapeDtypeStruct((NP, HID), jnp.float32),
        jax.ShapeDtypeStruct((NP, 128), jnp.float32),
    ],
)


def _tc_b_body(p_ref, hs1_ref, dinv_ref, b1_ref, w2_ref, hs2_ref):
  p = p_ref[...]
  dinv = dinv_ref[...]
  agg = p[0] + p[1] + hs1_ref[...]
  out1 = agg * dinv[:, :HID] + b1_ref[...]
  r = jnp.maximum(out1, 0.0)
  h2 = jnp.dot(r, w2_ref[...], preferred_element_type=jnp.float32)
  hs2_ref[...] = h2 * dinv[:, :OUT]


_tc_b = pl.pallas_call(
    _tc_b_body,
    grid=(NP // ROW_BLK,),
    in_specs=[
        pl.BlockSpec((NC, ROW_BLK, HID), lambda j: (0, j, 0)),
        pl.BlockSpec((ROW_BLK, HID), lambda j: (j, 0)),
        pl.BlockSpec((ROW_BLK, 128), lambda j: (j, 0)),
        pl.BlockSpec((1, HID), lambda j: (0, 0)),
        pl.BlockSpec((HID, OUT), lambda j: (0, 0)),
    ],
    out_specs=pl.BlockSpec((ROW_BLK, OUT), lambda j: (j, 0)),
    out_shape=jax.ShapeDtypeStruct((NP, OUT), jnp.float32),
)


def _tc_c_body(q_ref, hs2_ref, dinv_ref, b2_ref, out_ref):
  q = q_ref[...]
  dinv = dinv_ref[...]
  out_ref[...] = (q[0] + q[1] + hs2_ref[...]) * dinv[:, :OUT] + b2_ref[...]


_tc_c = pl.pallas_call(
    _tc_c_body,
    grid=(NP // ROW_BLK,),
    in_specs=[
        pl.BlockSpec((NC, ROW_BLK, OUT), lambda j: (0, j, 0)),
        pl.BlockSpec((ROW_BLK, OUT), lambda j: (j, 0)),
        pl.BlockSpec((ROW_BLK, 128), lambda j: (j, 0)),
        pl.BlockSpec((1, OUT), lambda j: (0, 0)),
    ],
    out_specs=pl.BlockSpec((ROW_BLK, OUT), lambda j: (j, 0)),
    out_shape=jax.ShapeDtypeStruct((NP, OUT), jnp.float32),
)


@jax.jit
def kernel(x, edge_index, W1, b1, W2, b2):
  ei = edge_index.astype(jnp.int32)
  src2 = ei[0].reshape(NW, RPT, EB)
  dst1 = ei[1]
  dst2 = dst1.reshape(NW, RPT, EB)

  x_p = jnp.pad(x, ((0, NP - N), (0, 0)))
  degp = _deg_call(dst1)
  hs1, dinv = _tc_a(x_p, W1, degp)
  p = _agg_hid(hs1, src2, dst2)
  hs2 = _tc_b(p, hs1, dinv, b1.reshape(1, HID), W2)
  q = _agg_out(hs2, src2, dst2)
  return _tc_c(q, hs2, dinv, b2.reshape(1, OUT))[:N]


# speedup vs baseline: 49.4002x; 1.1035x over previous
import functools

import jax
import jax.numpy as jnp
from jax import lax
from jax.experimental import pallas as pl
from jax.experimental.pallas import tpu as pltpu
from jax.experimental.pallas import tpu_sc as plsc

N = 10000
NP = 10240
E = 320000
IN_DIM = 128
HID = 64
OUT = 32

NC = 2
NS = 16
NW = NC * NS
EB = 125
EROWS = E // EB
RPT = EROWS // NW
GRP = 4
NIT = RPT // (2 * GRP)
NBUF = 2 * GRP
NODES_PER_TILE = NP // NS

ROW_BLK = 2048


def _mesh():
  return plsc.VectorSubcoreMesh(core_axis_name="c", subcore_axis_name="s")


def _deg_body(dst_hbm, degp_hbm, dst_v, deg_v):
  c = lax.axis_index("c")
  s = lax.axis_index("s")
  wid = s * NC + c
  per_tile = E // NW
  ch = 2000

  zeros16 = jnp.zeros((16,), jnp.float32)

  def zero_body(i, carry):
    deg_v[pl.ds(i * 16, 16)] = zeros16
    return carry

  lax.fori_loop(0, NP // 16, zero_body, 0)

  ones16 = jnp.ones((16,), jnp.float32)

  def chunk_body(ci, carry):
    base = wid * per_tile + ci * ch
    pltpu.sync_copy(dst_hbm.at[pl.ds(base, ch)], dst_v)

    def inner(i, c2):
      idx = dst_v[pl.ds(i * 16, 16)]
      plsc.addupdate_scatter(deg_v, [idx], ones16)
      return c2

    lax.fori_loop(0, ch // 16, inner, 0)
    return carry

  lax.fori_loop(0, per_tile // ch, chunk_body, 0)
  pltpu.sync_copy(deg_v, degp_hbm.at[wid])


_SC_PARAMS = pltpu.CompilerParams(needs_layout_passes=False,
                                  use_tc_tiling_on_sc=False)

_deg_call = pl.kernel(
    _deg_body,
    out_type=jax.ShapeDtypeStruct((NW, NP), jnp.float32),
    mesh=_mesh(),
    compiler_params=_SC_PARAMS,
    scratch_types=[
        pltpu.VMEM((2000,), jnp.int32),
        pltpu.VMEM((NP,), jnp.float32),
    ],
)


def _agg_body(hs_hbm, src_hbm, dst_hbm, aggp_hbm, idxs_v, idxd_v, rows_v,
              gsem_a, gsem_b, agg_sp, *, dim):
  c = lax.axis_index("c")
  s = lax.axis_index("s")
  wid = s * NC + c

  zeros16 = jnp.zeros((16,), jnp.float32)
  lanes_per_row = dim // 16

  def zero_body(i, carry):
    for l in range(lanes_per_row):
      rows_v[i, pl.ds(l * 16, 16)] = zeros16
    return carry

  pltpu.async_copy(src_hbm.at[wid], idxs_v, gsem_a)
  pltpu.async_copy(dst_hbm.at[wid], idxd_v, gsem_b)
  lax.fori_loop(0, NODES_PER_TILE, zero_body, 0)
  pltpu.sync_copy(rows_v.at[pl.ds(0, NODES_PER_TILE)],
                  agg_sp.at[pl.ds(s * NODES_PER_TILE, NODES_PER_TILE)])
  pltpu.make_async_copy(src_hbm.at[wid], idxs_v, gsem_a).wait()
  pltpu.make_async_copy(dst_hbm.at[wid], idxd_v, gsem_b).wait()
  plsc.subcore_barrier()

  def gather(j, b, sem):
    pltpu.async_copy(hs_hbm.at[idxs_v.at[j]], rows_v.at[pl.ds(b * EB, EB)], sem)

  def gwait(b, sem):
    pltpu.make_async_copy(hs_hbm.at[idxs_v.at[0]],
                          rows_v.at[pl.ds(b * EB, EB)], sem).wait()

  def scat(j, b):
    pltpu.sync_copy(rows_v.at[pl.ds(b * EB, EB)], agg_sp.at[idxd_v.at[j]],
                    add=True)

  for b in range(GRP):
    gather(b, b, gsem_a)

  def body(t, carry):
    g0 = 2 * t * GRP
    for b in range(GRP):
      gather(g0 + GRP + b, GRP + b, gsem_b)
    for b in range(GRP):
      gwait(b, gsem_a)
    for b in range(GRP):
      scat(g0 + b, b)
    @pl.when(t < NIT - 1)
    def _():
      for b in range(GRP):
        gather(g0 + 2 * GRP + b, b, gsem_a)
    for b in range(GRP):
      gwait(GRP + b, gsem_b)
    for b in range(GRP):
      scat(g0 + GRP + b, GRP + b)
    return carry

  lax.fori_loop(0, NIT, body, 0)
  plsc.subcore_barrier()
  pltpu.sync_copy(
      agg_sp.at[pl.ds(s * NODES_PER_TILE, NODES_PER_TILE)],
      aggp_hbm.at[c, pl.ds(s * NODES_PER_TILE, NODES_PER_TILE)],
  )


def _make_agg(dim):
  return pl.kernel(
      functools.partial(_agg_body, dim=dim),
      out_type=jax.ShapeDtypeStruct((NC, NP, dim), jnp.float32),
      mesh=_mesh(),
      compiler_params=_SC_PARAMS,
      scratch_types=[
          pltpu.VMEM((RPT, EB), jnp.int32),
          pltpu.VMEM((RPT, EB), jnp.int32),
          pltpu.VMEM((NBUF * EB, dim), jnp.float32),
          pltpu.SemaphoreType.DMA,
          pltpu.SemaphoreType.DMA,
          pltpu.VMEM_SHARED((NP, dim), jnp.float32),
      ],
  )


_agg_hid = _make_agg(HID)
_agg_out = _make_agg(OUT)


def _tc_a_body(x_ref, w1_ref, degp_ref, hs1_ref, dinv_ref):
  degp = degp_ref[...]
  deg = jnp.sum(degp.T, axis=1, keepdims=True) + 1.0
  dinv = lax.rsqrt(deg)
  h = jnp.dot(x_ref[...], w1_ref[...], preferred_element_type=jnp.float32)
  hs1_ref[...] = h * dinv
  dinv_ref[...] = jnp.broadcast_to(dinv, dinv_ref.shape)


_tc_a = pl.pallas_call(
    _tc_a_body,
    grid=(NP // ROW_BLK,),
    in_specs=[
        pl.BlockSpec((ROW_BLK, IN_DIM), lambda j: (j, 0)),
        pl.BlockSpec((IN_DIM, HID), lambda j: (0, 0)),
        pl.BlockSpec((NW, ROW_BLK), lambda j: (0, j)),
    ],
    out_specs=[
        pl.BlockSpec((ROW_BLK, HID), lambda j: (j, 0)),
        pl.BlockSpec((ROW_BLK, 128), lambda j: (j, 0)),
    ],
    out_shape=[
        jax.ShapeDtypeStruct((NP, HID), jnp.float32),
        jax.ShapeDtypeStruct((NP, 128), jnp.float32),
    ],
)


def _tc_b_body(p_ref, hs1_ref, dinv_ref, b1_ref, w2_ref, hs2_ref):
  p = p_ref[...]
  dinv = dinv_ref[...]
  agg = p[0] + p[1] + hs1_ref[...]
  out1 = agg * dinv[:, :HID] + b1_ref[...]
  r = jnp.maximum(out1, 0.0)
  h2 = jnp.dot(r, w2_ref[...], preferred_element_type=jnp.float32)
  hs2_ref[...] = h2 * dinv[:, :OUT]


_tc_b = pl.pallas_call(
    _tc_b_body,
    grid=(NP // ROW_BLK,),
    in_specs=[
        pl.BlockSpec((NC, ROW_BLK, HID), lambda j: (0, j, 0)),
        pl.BlockSpec((ROW_BLK, HID), lambda j: (j, 0)),
        pl.BlockSpec((ROW_BLK, 128), lambda j: (j, 0)),
        pl.BlockSpec((1, HID), lambda j: (0, 0)),
        pl.BlockSpec((HID, OUT), lambda j: (0, 0)),
    ],
    out_specs=pl.BlockSpec((ROW_BLK, OUT), lambda j: (j, 0)),
    out_shape=jax.ShapeDtypeStruct((NP, OUT), jnp.float32),
)


def _tc_c_body(q_ref, hs2_ref, dinv_ref, b2_ref, out_ref):
  q = q_ref[...]
  dinv = dinv_ref[...]
  out_ref[...] = (q[0] + q[1] + hs2_ref[...]) * dinv[:, :OUT] + b2_ref[...]


_tc_c = pl.pallas_call(
    _tc_c_body,
    grid=(NP // ROW_BLK,),
    in_specs=[
        pl.BlockSpec((NC, ROW_BLK, OUT), lambda j: (0, j, 0)),
        pl.BlockSpec((ROW_BLK, OUT), lambda j: (j, 0)),
        pl.BlockSpec((ROW_BLK, 128), lambda j: (j, 0)),
        pl.BlockSpec((1, OUT), lambda j: (0, 0)),
    ],
    out_specs=pl.BlockSpec((ROW_BLK, OUT), lambda j: (j, 0)),
    out_shape=jax.ShapeDtypeStruct((NP, OUT), jnp.float32),
)


@jax.jit
def kernel(x, edge_index, W1, b1, W2, b2):
  ei = edge_index.astype(jnp.int32)
  src2 = ei[0].reshape(NW, RPT, EB)
  dst1 = ei[1]
  dst2 = dst1.reshape(NW, RPT, EB)

  x_p = jnp.pad(x, ((0, NP - N), (0, 0)))
  degp = _deg_call(dst1)
  hs1, dinv = _tc_a(x_p, W1, degp)
  p = _agg_hid(hs1, src2, dst2)
  hs2 = _tc_b(p, hs1, dinv, b1.reshape(1, HID), W2)
  q = _agg_out(hs2, src2, dst2)
  return _tc_c(q, hs2, dinv, b2.reshape(1, OUT))[:N]
